# Initial kernel scaffold; baseline (speedup 1.0000x reference)
import jax
import jax.numpy as jnp
from jax.experimental import pallas as pl
from jax.experimental.pallas import tpu as pltpu


def _add_body(months_ref, tok_ref, ch_ref, pos_ref, mon_ref, out_ref):
    del months_ref
    bs, n = ch_ref.shape
    ch = ch_ref[...]
    pe = jnp.broadcast_to(pos_ref[...], (bs, n))
    me = jnp.broadcast_to(mon_ref[...], (bs, n))
    z = jnp.zeros((bs, n), jnp.float32)
    add = jnp.concatenate([ch, pe, me, z], axis=-1)
    out_ref[...] = tok_ref[...] + add[None, None, None, None, :, :]


def kernel(tokens, timestamps, channel_embed, pos_embed, month_table):
    b, h, w, t, bs, d = tokens.shape
    n = d // 4
    months = timestamps[:, :, 1].astype(jnp.int32)

    grid_spec = pltpu.PrefetchScalarGridSpec(
        num_scalar_prefetch=1,
        grid=(b, t),
        in_specs=[
            pl.BlockSpec((1, h, w, 1, bs, d), lambda i, j, m: (i, 0, 0, j, 0, 0)),
            pl.BlockSpec((bs, n), lambda i, j, m: (0, 0)),
            pl.BlockSpec((1, n), lambda i, j, m: (j, 0)),
            pl.BlockSpec((1, n), lambda i, j, m: (m[i, j], 0)),
        ],
        out_specs=pl.BlockSpec((1, h, w, 1, bs, d), lambda i, j, m: (i, 0, 0, j, 0, 0)),
    )

    return pl.pallas_call(
        _add_body,
        grid_spec=grid_spec,
        out_shape=jax.ShapeDtypeStruct(tokens.shape, tokens.dtype),
    )(months, tokens, channel_embed, pos_embed, month_table)


# speedup vs baseline: 2.5582x; 2.5582x over previous
import jax
import jax.numpy as jnp
from jax.experimental import pallas as pl
from jax.experimental.pallas import tpu as pltpu


def _add_body(months_ref, tok_ref, ch_ref, pos_ref, mon_ref, out_ref):
    del months_ref
    bs, n = ch_ref.shape
    ch = ch_ref[...]
    pe = jnp.broadcast_to(pos_ref[0], (bs, n))
    me = jnp.broadcast_to(mon_ref[0], (bs, n))
    z = jnp.zeros((bs, n), jnp.float32)
    add = jnp.concatenate([ch, pe, me, z], axis=-1)
    out_ref[...] = tok_ref[...] + add[None, None, None, None, :, :]


def kernel(tokens, timestamps, channel_embed, pos_embed, month_table):
    b, h, w, t, bs, d = tokens.shape
    n = d // 4
    months = timestamps[:, :, 1].astype(jnp.int32)
    pos3 = pos_embed.reshape(pos_embed.shape[0], 1, n)
    mon3 = month_table.reshape(month_table.shape[0], 1, n)

    grid_spec = pltpu.PrefetchScalarGridSpec(
        num_scalar_prefetch=1,
        grid=(b, t),
        in_specs=[
            pl.BlockSpec((1, h, w, 1, bs, d), lambda i, j, m: (i, 0, 0, j, 0, 0)),
            pl.BlockSpec((bs, n), lambda i, j, m: (0, 0)),
            pl.BlockSpec((1, 1, n), lambda i, j, m: (j, 0, 0)),
            pl.BlockSpec((1, 1, n), lambda i, j, m: (m[i, j], 0, 0)),
        ],
        out_specs=pl.BlockSpec((1, h, w, 1, bs, d), lambda i, j, m: (i, 0, 0, j, 0, 0)),
    )

    return pl.pallas_call(
        _add_body,
        grid_spec=grid_spec,
        out_shape=jax.ShapeDtypeStruct(tokens.shape, tokens.dtype),
    )(months, tokens, channel_embed, pos3, mon3)
